# baseline (speedup 1.0000x reference)
import functools

import jax
import jax.numpy as jnp
from jax import lax
from jax.experimental import pallas as pl
from jax.experimental.pallas import tpu as pltpu
from jax.experimental.pallas import tpu_sc as plsc

_VOCAB = 1000000
_D = 64
_B = 4096
_S = 200
_SB = _B * _S

try:
    _info = plsc.get_sparse_core_info()
    _NC, _NS = _info.num_cores, _info.num_subcores
except Exception:
    _NC, _NS = 2, 16
_NW = _NC * _NS
_GBLK = 128
_ROWS_PER_W = _SB // _NW
_BLOCKS_PER_W = _ROWS_PER_W // _GBLK

_mesh = plsc.VectorSubcoreMesh(core_axis_name="c", subcore_axis_name="s")


@functools.partial(
    pl.kernel,
    mesh=_mesh,
    out_type=jax.ShapeDtypeStruct((_SB, _D), jnp.float32),
    compiler_params=pltpu.CompilerParams(use_tc_tiling_on_sc=False),
    scratch_types=[
        pltpu.VMEM((_BLOCKS_PER_W, _GBLK), jnp.int32),
        pltpu.VMEM((_GBLK, _D), jnp.float32),
        pltpu.SemaphoreType.DMA,
    ],
)
def _sc_gather(idx_hbm, table_hbm, out_hbm, idx_v, rows_v, sem):
    wid = lax.axis_index("s") * _NC + lax.axis_index("c")
    blk_base = wid * _BLOCKS_PER_W
    pltpu.sync_copy(idx_hbm.at[pl.ds(blk_base, _BLOCKS_PER_W)], idx_v)

    def body(j, carry):
        pltpu.async_copy(table_hbm.at[idx_v.at[j]], rows_v, sem).wait()
        pltpu.sync_copy(rows_v, out_hbm.at[pl.ds((blk_base + j) * _GBLK, _GBLK)])
        return carry

    lax.fori_loop(0, _BLOCKS_PER_W, body, 0)


def _mask_body(x_ref, o_ref):
    o_ref[...] = (x_ref[...] != 0).astype(jnp.float32)


_tc_mask = pl.pallas_call(
    _mask_body,
    out_shape=jax.ShapeDtypeStruct((_B, _S), jnp.float32),
)


def kernel(x, table):
    x = x.astype(jnp.int32)
    idx_t = x.T.reshape(_SB // _GBLK, _GBLK)
    out_flat = _sc_gather(idx_t, table)
    out_emb = out_flat.reshape(_S, _B, _D)
    mask = _tc_mask(x)
    return (out_emb, mask)


# speedup vs baseline: 1.1189x; 1.1189x over previous
import functools

import jax
import jax.numpy as jnp
from jax import lax
from jax.experimental import pallas as pl
from jax.experimental.pallas import tpu as pltpu
from jax.experimental.pallas import tpu_sc as plsc

_VOCAB = 1000000
_D = 64
_B = 4096
_S = 200
_SB = _B * _S

try:
    _info = plsc.get_sparse_core_info()
    _NC, _NS = _info.num_cores, _info.num_subcores
except Exception:
    _NC, _NS = 2, 16
_NW = _NC * _NS
_GBLK = 128
_ROWS_PER_W = _SB // _NW
_BLOCKS_PER_W = _ROWS_PER_W // _GBLK

_mesh = plsc.VectorSubcoreMesh(core_axis_name="c", subcore_axis_name="s")


_CHUNK = 512
_GPC = _CHUNK // _GBLK
_NCHUNK = _ROWS_PER_W // _CHUNK


@functools.partial(
    pl.kernel,
    mesh=_mesh,
    out_type=jax.ShapeDtypeStruct((_SB, _D), jnp.float32),
    compiler_params=pltpu.CompilerParams(use_tc_tiling_on_sc=False),
    scratch_types=[
        pltpu.VMEM((_BLOCKS_PER_W, _GBLK), jnp.int32),
        pltpu.VMEM((_CHUNK, _D), jnp.float32),
        pltpu.VMEM((_CHUNK, _D), jnp.float32),
        pltpu.SemaphoreType.DMA,
        pltpu.SemaphoreType.DMA,
        pltpu.SemaphoreType.DMA,
        pltpu.SemaphoreType.DMA,
    ],
)
def _sc_gather(idx_hbm, table_hbm, out_hbm, idx_v, rows_a, rows_b, gsa, gsb, osa, osb):
    wid = lax.axis_index("s") * _NC + lax.axis_index("c")
    blk_base = wid * _BLOCKS_PER_W
    row_base = wid * _ROWS_PER_W
    pltpu.sync_copy(idx_hbm.at[pl.ds(blk_base, _BLOCKS_PER_W)], idx_v)

    def gathers(c, rows_v, sem):
        for k in range(_GPC):
            pltpu.async_copy(
                table_hbm.at[idx_v.at[c * _GPC + k]],
                rows_v.at[pl.ds(k * _GBLK, _GBLK)],
                sem,
            )

    def wait_gathers(c, rows_v, sem):
        for k in range(_GPC):
            pltpu.make_async_copy(
                table_hbm.at[idx_v.at[c * _GPC + k]],
                rows_v.at[pl.ds(k * _GBLK, _GBLK)],
                sem,
            ).wait()

    def out_copy(c, rows_v, sem):
        pltpu.async_copy(rows_v, out_hbm.at[pl.ds(row_base + c * _CHUNK, _CHUNK)], sem)

    def wait_out(rows_v, sem):
        pltpu.make_async_copy(rows_v, out_hbm.at[pl.ds(row_base, _CHUNK)], sem).wait()

    gathers(0, rows_a, gsa)

    def body(t, carry):
        c0 = 2 * t

        @pl.when(t > 0)
        def _():
            wait_out(rows_b, osb)

        gathers(c0 + 1, rows_b, gsb)
        wait_gathers(c0, rows_a, gsa)
        out_copy(c0, rows_a, osa)

        @pl.when(t < _NCHUNK // 2 - 1)
        def _():
            wait_out(rows_a, osa)
            gathers(c0 + 2, rows_a, gsa)

        wait_gathers(c0 + 1, rows_b, gsb)
        out_copy(c0 + 1, rows_b, osb)
        return carry

    lax.fori_loop(0, _NCHUNK // 2, body, 0)
    wait_out(rows_a, osa)
    wait_out(rows_b, osb)


def _mask_body(x_ref, o_ref):
    o_ref[...] = (x_ref[...] != 0).astype(jnp.float32)


_tc_mask = pl.pallas_call(
    _mask_body,
    out_shape=jax.ShapeDtypeStruct((_B, _S), jnp.float32),
)


def kernel(x, table):
    x = x.astype(jnp.int32)
    idx_t = x.T.reshape(_SB // _GBLK, _GBLK)
    out_flat = _sc_gather(idx_t, table)
    out_emb = out_flat.reshape(_S, _B, _D)
    mask = _tc_mask(x)
    return (out_emb, mask)
